# Initial kernel scaffold; baseline (speedup 1.0000x reference)
import jax
import jax.numpy as jnp
from jax.experimental import pallas as pl


def kernel(params, dino_feat, pos, smpl, mat, edge_attr, edge_index, batch):
    raise NotImplementedError("write your pallas kernel here")


# speedup vs baseline: 2.0631x; 2.0631x over previous
import functools

import jax
import jax.numpy as jnp
from jax import lax
from jax.experimental import pallas as pl
from jax.experimental.pallas import tpu as pltpu
from jax.experimental.pallas import tpu_sc as plsc

D = 128
NW = 32
CH = 128
TILE_E = 2048
TILE_N = 2000


def _ln_relu(h, g, b):
    m = jnp.mean(h, axis=-1, keepdims=True)
    v = jnp.mean(jnp.square(h - m), axis=-1, keepdims=True)
    return jnp.maximum((h - m) * lax.rsqrt(v + 1e-5) * g + b, 0.0)


def _mlp3_body(x_ref, w1, b1, g1, be1, w2, b2, g2, be2, w3, b3, o_ref):
    h = jnp.dot(x_ref[...], w1[...], preferred_element_type=jnp.float32) + b1[...]
    h = _ln_relu(h, g1[...], be1[...])
    h = jnp.dot(h, w2[...], preferred_element_type=jnp.float32) + b2[...]
    h = _ln_relu(h, g2[...], be2[...])
    o_ref[...] = jnp.dot(h, w3[...], preferred_element_type=jnp.float32) + b3[...]


def _edge_body(xs_ref, xd_ref, e_ref, w1a, w1b, w1c, b1, g1, be1,
               w2, b2, g2, be2, w3, b3, o_ref):
    e = e_ref[...]
    h = (jnp.dot(xs_ref[...], w1a[...], preferred_element_type=jnp.float32)
         + jnp.dot(xd_ref[...], w1b[...], preferred_element_type=jnp.float32)
         + jnp.dot(e, w1c[...], preferred_element_type=jnp.float32) + b1[...])
    h = _ln_relu(h, g1[...], be1[...])
    h = jnp.dot(h, w2[...], preferred_element_type=jnp.float32) + b2[...]
    h = _ln_relu(h, g2[...], be2[...])
    o_ref[...] = e + jnp.dot(h, w3[...], preferred_element_type=jnp.float32) + b3[...]


def _node_body(x_ref, m_ref, w1x, w1m, b1, g1, be1,
               w2, b2, g2, be2, w3, b3, o_ref):
    x = x_ref[...]
    msg = m_ref[0] + m_ref[1]
    h = (jnp.dot(x, w1x[...], preferred_element_type=jnp.float32)
         + jnp.dot(msg, w1m[...], preferred_element_type=jnp.float32) + b1[...])
    h = _ln_relu(h, g1[...], be1[...])
    h = jnp.dot(h, w2[...], preferred_element_type=jnp.float32) + b2[...]
    h = _ln_relu(h, g2[...], be2[...])
    o_ref[...] = x + jnp.dot(h, w3[...], preferred_element_type=jnp.float32) + b3[...]


def _style_body(d_ref, w1, b1, w2, b2, o_ref):
    h = jnp.dot(d_ref[...], w1[...], preferred_element_type=jnp.float32) + b1[...]
    h = 0.5 * h * (1.0 + lax.erf(h / jnp.sqrt(2.0).astype(jnp.float32)))
    o_ref[...] = jnp.dot(h, w2[...], preferred_element_type=jnp.float32) + b2[...]


def _dec_body(x_ref, w, b, o_ref):
    o_ref[...] = jnp.dot(x_ref[...], w[...], preferred_element_type=jnp.float32) + b[...]


def _full_spec(arr):
    return pl.BlockSpec(arr.shape, lambda i: (0,) * arr.ndim)


def _row_spec(tile, k):
    return pl.BlockSpec((tile, k), lambda i: (i, 0))


def _mlp3(x, p, tile):
    rows, k = x.shape
    out_dim = p["W3"].shape[1]
    ws = [p["W1"], p["b1"].reshape(1, -1), p["g1"].reshape(1, -1),
          p["be1"].reshape(1, -1), p["W2"], p["b2"].reshape(1, -1),
          p["g2"].reshape(1, -1), p["be2"].reshape(1, -1), p["W3"],
          p["b3"].reshape(1, -1)]
    return pl.pallas_call(
        _mlp3_body,
        grid=(rows // tile,),
        in_specs=[_row_spec(tile, k)] + [_full_spec(w) for w in ws],
        out_specs=_row_spec(tile, out_dim),
        out_shape=jax.ShapeDtypeStruct((rows, out_dim), jnp.float32),
    )(x, *ws)


def _edge_mlp(xs, xd, e, p):
    rows = xs.shape[0]
    w1a, w1b, w1c = p["W1"][:D], p["W1"][D:2 * D], p["W1"][2 * D:]
    ws = [w1a, w1b, w1c, p["b1"].reshape(1, -1), p["g1"].reshape(1, -1),
          p["be1"].reshape(1, -1), p["W2"], p["b2"].reshape(1, -1),
          p["g2"].reshape(1, -1), p["be2"].reshape(1, -1), p["W3"],
          p["b3"].reshape(1, -1)]
    return pl.pallas_call(
        _edge_body,
        grid=(rows // TILE_E,),
        in_specs=[_row_spec(TILE_E, D)] * 3 + [_full_spec(w) for w in ws],
        out_specs=_row_spec(TILE_E, D),
        out_shape=jax.ShapeDtypeStruct((rows, D), jnp.float32),
    )(xs, xd, e, *ws)


def _node_mlp(x, msgp, p):
    rows = x.shape[0]
    w1x, w1m = p["W1"][:D], p["W1"][D:]
    ws = [w1x, w1m, p["b1"].reshape(1, -1), p["g1"].reshape(1, -1),
          p["be1"].reshape(1, -1), p["W2"], p["b2"].reshape(1, -1),
          p["g2"].reshape(1, -1), p["be2"].reshape(1, -1), p["W3"],
          p["b3"].reshape(1, -1)]
    return pl.pallas_call(
        _node_body,
        grid=(rows // TILE_N,),
        in_specs=[_row_spec(TILE_N, D),
                  pl.BlockSpec((2, TILE_N, D), lambda i: (0, i, 0))]
                 + [_full_spec(w) for w in ws],
        out_specs=_row_spec(TILE_N, D),
        out_shape=jax.ShapeDtypeStruct((rows, D), jnp.float32),
    )(x, msgp, *ws)


def _style_proj(dino, w1, b1, w2, b2):
    ws = [w1, b1.reshape(1, -1), w2, b2.reshape(1, -1)]
    return pl.pallas_call(
        _style_body,
        grid=(1,),
        in_specs=[_full_spec(dino)] + [_full_spec(w) for w in ws],
        out_specs=pl.BlockSpec((dino.shape[0], D), lambda i: (0, 0)),
        out_shape=jax.ShapeDtypeStruct((dino.shape[0], D), jnp.float32),
    )(dino, *ws)


def _decode(x, w, b):
    rows = x.shape[0]
    wp = jnp.pad(w, ((0, 0), (0, D - w.shape[1])))
    bp = jnp.pad(b, (0, D - b.shape[0])).reshape(1, -1)
    out = pl.pallas_call(
        _dec_body,
        grid=(rows // TILE_N,),
        in_specs=[_row_spec(TILE_N, D), _full_spec(wp), _full_spec(bp)],
        out_specs=_row_spec(TILE_N, D),
        out_shape=jax.ShapeDtypeStruct((rows, D), jnp.float32),
    )(x, wp, bp)
    return out[:, :w.shape[1]]


def _make_gather(n, kj):
    e_pad = NW * kj * CH
    mesh = plsc.VectorSubcoreMesh(core_axis_name="c", subcore_axis_name="s")

    @functools.partial(
        pl.kernel, mesh=mesh,
        out_type=[jax.ShapeDtypeStruct((e_pad, D), jnp.float32),
                  jax.ShapeDtypeStruct((e_pad, D), jnp.float32)],
        scratch_types=[
            pltpu.VMEM((kj, CH), jnp.int32),
            pltpu.VMEM((kj, CH), jnp.int32),
            pltpu.VMEM((CH, D), jnp.float32),
            pltpu.VMEM((CH, D), jnp.float32),
            pltpu.SemaphoreType.DMA,
            pltpu.SemaphoreType.DMA,
        ],
    )
    def gather2(x_hbm, row_hbm, col_hbm, xs_hbm, xd_hbm,
                ridx, cidx, rbuf, cbuf, rsem, csem):
        c = lax.axis_index("c")
        s = lax.axis_index("s")
        wid = s * 2 + c
        pltpu.sync_copy(row_hbm.at[wid], ridx)
        pltpu.sync_copy(col_hbm.at[wid], cidx)
        base = wid * kj * CH

        def body(j, carry):
            pltpu.async_copy(x_hbm.at[ridx.at[j]], rbuf, rsem).wait()
            pltpu.sync_copy(rbuf, xs_hbm.at[pl.ds(base + j * CH, CH)])
            pltpu.async_copy(x_hbm.at[cidx.at[j]], cbuf, csem).wait()
            pltpu.sync_copy(cbuf, xd_hbm.at[pl.ds(base + j * CH, CH)])
            return carry

        lax.fori_loop(0, kj, body, 0)

    return gather2


def _make_scatter(n_acc, kj):
    mesh = plsc.VectorSubcoreMesh(core_axis_name="c", subcore_axis_name="s")
    rows_per_sub = n_acc // 16

    @functools.partial(
        pl.kernel, mesh=mesh,
        out_type=jax.ShapeDtypeStruct((2, n_acc, D), jnp.float32),
        scratch_types=[
            pltpu.VMEM((kj, CH), jnp.int32),
            pltpu.VMEM((CH, D), jnp.float32),
            pltpu.VMEM_SHARED((n_acc, D), jnp.float32),
        ],
    )
    def scatter_add(e_hbm, col_hbm, zeros_hbm, out_hbm, cidx, ebuf, acc):
        c = lax.axis_index("c")
        s = lax.axis_index("s")
        wid = s * 2 + c
        pltpu.sync_copy(zeros_hbm.at[pl.ds(s * rows_per_sub, rows_per_sub)],
                        acc.at[pl.ds(s * rows_per_sub, rows_per_sub)])
        plsc.subcore_barrier()
        pltpu.sync_copy(col_hbm.at[wid], cidx)
        base = wid * kj * CH

        def body(j, carry):
            pltpu.sync_copy(e_hbm.at[pl.ds(base + j * CH, CH)], ebuf)
            pltpu.sync_copy(ebuf, acc.at[cidx.at[j]], add=True)
            return carry

        lax.fori_loop(0, kj, body, 0)
        plsc.subcore_barrier()
        pltpu.sync_copy(acc.at[pl.ds(s * rows_per_sub, rows_per_sub)],
                        out_hbm.at[c, pl.ds(s * rows_per_sub, rows_per_sub)])

    return scatter_add


def kernel(params, dino_feat, pos, smpl, mat, edge_attr, edge_index, batch):
    n = pos.shape[0]
    e_cnt = edge_attr.shape[0]
    kj = -(-e_cnt // (NW * CH))
    e_pad = NW * kj * CH
    n_acc = -(-(n + 1) // 128) * 128

    style = _style_proj(dino_feat, params["proj_W1"], params["proj_b1"],
                        params["proj_W2"], params["proj_b2"])

    x_in = jnp.concatenate(
        [pos, style[batch], smpl[batch], mat[batch]], axis=-1)
    k_in = x_in.shape[1]
    k_pad = -(-k_in // 8) * 8
    x_in = jnp.pad(x_in, ((0, 0), (0, k_pad - k_in)))
    ne = dict(params["node_enc"])
    ne["W1"] = jnp.pad(ne["W1"], ((0, k_pad - k_in), (0, 0)))
    x = _mlp3(x_in, ne, TILE_N)

    ea_pad = jnp.pad(edge_attr, ((0, e_pad - e_cnt), (0, 0)))
    e = _mlp3(ea_pad, params["edge_enc"], TILE_E)

    row = edge_index[0]
    col = edge_index[1]
    pad = e_pad - e_cnt
    row_g = jnp.pad(row, (0, pad)).reshape(NW, kj, CH)
    col_g = jnp.pad(col, (0, pad)).reshape(NW, kj, CH)
    col_s = jnp.pad(col, (0, pad), constant_values=n).reshape(NW, kj, CH)
    zeros_acc = jnp.zeros((n_acc, D), jnp.float32)

    gather2 = _make_gather(n, kj)
    scatter_add = _make_scatter(n_acc, kj)

    for blk in params["blocks"]:
        xs, xd = gather2(x, row_g, col_g)
        e = _edge_mlp(xs, xd, e, blk["edge_mlp"])
        msgp = scatter_add(e, col_s, zeros_acc)
        x = _node_mlp(x, msgp, blk["node_mlp"])

    return _decode(x, params["dec_W"], params["dec_b"])
